# baseline (speedup 1.0000x reference)
import functools

import jax
import jax.numpy as jnp
from jax import lax
from jax.experimental import pallas as pl
from jax.experimental.pallas import tpu as pltpu
from jax.experimental.pallas import tpu_sc as plsc


def _grab_units_sc(x_flat, chans, row, col, *, B, U, CHW, HW, W):
    info = plsc.get_sparse_core_info()
    nw = info.num_cores * info.num_subcores
    per = B // nw
    mesh = plsc.VectorSubcoreMesh(core_axis_name="c", subcore_axis_name="s")

    @functools.partial(
        pl.kernel,
        mesh=mesh,
        out_type=jax.ShapeDtypeStruct((B, U), jnp.float32),
        scratch_types=[
            pltpu.VMEM((U,), jnp.int32),
            pltpu.VMEM((U,), jnp.int32),
            pltpu.VMEM((U,), jnp.int32),
            pltpu.VMEM((U,), jnp.int32),
            pltpu.VMEM((U,), jnp.float32),
            pltpu.SemaphoreType.DMA,
        ],
    )
    def k(x_hbm, chans_hbm, row_hbm, col_hbm, out_hbm, ch_v, r_v, c_v, idx_v, vals_v, sem):
        wid = lax.axis_index("s") * info.num_cores + lax.axis_index("c")
        pltpu.sync_copy(chans_hbm, ch_v)
        pltpu.sync_copy(row_hbm, r_v)
        pltpu.sync_copy(col_hbm, c_v)
        for lb in range(per):
            b = wid * per + lb
            boff = b * CHW
            for i in range(U // 16):
                s = pl.ds(i * 16, 16)
                idx_v[s] = ch_v[s] * HW + r_v[s] * W + c_v[s] + boff
            pltpu.async_copy(x_hbm.at[idx_v], vals_v, sem).wait()
            pltpu.sync_copy(vals_v, out_hbm.at[b])

    return k(x_flat, chans, row, col)


def kernel(x, chans, coords):
    B, C, H, W = x.shape
    U = chans.shape[0]
    x_flat = x.reshape(-1)
    ch = chans.astype(jnp.int32)
    r = coords[:, 0].astype(jnp.int32)
    c = coords[:, 1].astype(jnp.int32)
    return _grab_units_sc(x_flat, ch, r, c, B=B, U=U, CHW=C * H * W, HW=H * W, W=W)


# speedup vs baseline: 2.1314x; 2.1314x over previous
import jax
import jax.numpy as jnp
from jax.experimental import pallas as pl
from jax.experimental.pallas import tpu as pltpu


def _grab_units(x, chans, rows, cols, rows_v, cols_v):
    B, C, H, W = x.shape
    U = chans.shape[0]
    CW = 128
    CH = 8

    def body(chans_ref, rows_ref, cols_ref, rowsv_ref, colsv_ref, x_ref,
             out_ref, buf, sem):
        for u in range(U):
            c = chans_ref[u]
            r_al = pl.multiple_of((rows_ref[u] // CH) * CH, CH)
            w_al = pl.multiple_of((cols_ref[u] // CW) * CW, CW)
            pltpu.make_async_copy(
                x_ref.at[:, c, pl.ds(r_al, CH), pl.ds(w_al, CW)],
                buf.at[:, u],
                sem,
            ).start()
        for u in range(U):
            pltpu.make_async_copy(
                x_ref.at[:, 0, pl.ds(0, CH), pl.ds(0, CW)],
                buf.at[:, u],
                sem,
            ).wait()
        rv = rowsv_ref[...]
        wv = colsv_ref[...]
        rm = rv % CH
        wm = wv % CW
        j_idx = jax.lax.broadcasted_iota(jnp.int32, (U, CH, CW), 1)
        l_idx = jax.lax.broadcasted_iota(jnp.int32, (U, CH, CW), 2)
        mask = (j_idx == rm[:, None, None]) & (l_idx == wm[:, None, None])
        masked = jnp.where(mask[None, :, :, :], buf[...], 0.0)
        out_ref[...] = jnp.sum(masked, axis=(2, 3))

    return pl.pallas_call(
        body,
        in_specs=[
            pl.BlockSpec(memory_space=pltpu.MemorySpace.SMEM),
            pl.BlockSpec(memory_space=pltpu.MemorySpace.SMEM),
            pl.BlockSpec(memory_space=pltpu.MemorySpace.SMEM),
            pl.BlockSpec(memory_space=pltpu.MemorySpace.VMEM),
            pl.BlockSpec(memory_space=pltpu.MemorySpace.VMEM),
            pl.BlockSpec(memory_space=pltpu.MemorySpace.HBM),
        ],
        out_specs=pl.BlockSpec(memory_space=pltpu.MemorySpace.VMEM),
        out_shape=jax.ShapeDtypeStruct((B, U), jnp.float32),
        scratch_shapes=[
            pltpu.VMEM((B, U, CH, CW), jnp.float32),
            pltpu.SemaphoreType.DMA,
        ],
    )(chans, rows, cols, rows_v, cols_v, x)


def kernel(x, chans, coords):
    ch = chans.astype(jnp.int32)
    r = coords[:, 0].astype(jnp.int32)
    c = coords[:, 1].astype(jnp.int32)
    return _grab_units(x, ch, r, c, r, c)
